# baseline (speedup 1.0000x reference)
import functools

import jax
import jax.numpy as jnp
from jax.experimental import pallas as pl
from jax.experimental.pallas import tpu as pltpu

Q = 1024
D = 128
KB = 10000
NBLK = 10

_DOT_PREC = jax.lax.Precision.DEFAULT


def _knn_block(qT_ref, d_ref, lab_ref, out_ref, m0_ref, m1_ref, *, nblk):
    j = pl.program_id(0)

    @pl.when(j == 0)
    def _init():
        m0_ref[...] = jnp.full_like(m0_ref, jnp.inf)
        m1_ref[...] = jnp.full_like(m1_ref, jnp.inf)

    d = d_ref[...]
    ej = (jax.lax.broadcasted_iota(jnp.int32, (1, NBLK), 1) == j)
    lab = jnp.sum(lab_ref[...] * ej.astype(jnp.float32), axis=1, keepdims=True)
    dot = jax.lax.dot(d, qT_ref[...], precision=_DOT_PREC)
    d2 = jnp.sum(d * d, axis=1, keepdims=True)
    col0 = jnp.where(lab == 0.0, d2, jnp.inf)
    col1 = jnp.where(lab == 1.0, d2, jnp.inf)
    m0_ref[...] = jnp.minimum(m0_ref[...],
                              jnp.min(dot + col0, axis=0, keepdims=True))
    m1_ref[...] = jnp.minimum(m1_ref[...],
                              jnp.min(dot + col1, axis=0, keepdims=True))

    @pl.when(j == nblk - 1)
    def _finish():
        out_ref[...] = jnp.where(m1_ref[...] < m0_ref[...], 1.0, 0.0)


@jax.jit
def kernel(input, data, labels):
    k = data.shape[0]
    assert k % KB == 0
    nblk = k // KB
    lab_p = labels.reshape(nblk, KB).T
    qT = input.T * jnp.float32(-2.0)

    pred_row = pl.pallas_call(
        functools.partial(_knn_block, nblk=nblk),
        grid=(nblk,),
        in_specs=[
            pl.BlockSpec((D, Q), lambda j: (0, 0)),
            pl.BlockSpec((KB, D), lambda j: (j, 0)),
            pl.BlockSpec((KB, NBLK), lambda j: (0, 0)),
        ],
        out_specs=pl.BlockSpec((1, Q), lambda j: (0, 0)),
        out_shape=jax.ShapeDtypeStruct((1, Q), jnp.float32),
        scratch_shapes=[
            pltpu.VMEM((1, Q), jnp.float32),
            pltpu.VMEM((1, Q), jnp.float32),
        ],
    )(qT, data, lab_p)

    return (pred_row.reshape(Q, 1), jnp.asarray(0.0))


# speedup vs baseline: 1.0555x; 1.0555x over previous
import functools

import jax
import jax.numpy as jnp
from jax.experimental import pallas as pl
from jax.experimental.pallas import tpu as pltpu

Q = 1024
D = 128
KB = 10000
NBLK = 10

_DOT_PREC = jax.lax.Precision.DEFAULT


def _knn_block(qT_ref, d_ref, lab_ref, out_ref, m0_ref, ma_ref, *, nblk):
    j = pl.program_id(0)

    @pl.when(j == 0)
    def _init():
        m0_ref[...] = jnp.full_like(m0_ref, jnp.inf)
        ma_ref[...] = jnp.full_like(ma_ref, jnp.inf)

    d = d_ref[...]
    lab = lab_ref[...].reshape(KB, 1)
    dot = jax.lax.dot(d, qT_ref[...], precision=_DOT_PREC)
    d2 = jnp.sum(d * d, axis=1, keepdims=True)
    col0 = jnp.where(lab == 0.0, d2, jnp.inf)
    m0_ref[...] = jnp.minimum(m0_ref[...],
                              jnp.min(dot + col0, axis=0, keepdims=True))
    ma_ref[...] = jnp.minimum(ma_ref[...],
                              jnp.min(dot + d2, axis=0, keepdims=True))

    @pl.when(j == nblk - 1)
    def _finish():
        out_ref[...] = jnp.where(ma_ref[...] < m0_ref[...], 1.0, 0.0)


@jax.jit
def kernel(input, data, labels):
    k = data.shape[0]
    assert k % KB == 0
    nblk = k // KB
    lab_p = labels.reshape(nblk, 1, KB)
    qT = input.T * jnp.float32(-2.0)

    pred_row = pl.pallas_call(
        functools.partial(_knn_block, nblk=nblk),
        grid=(nblk,),
        in_specs=[
            pl.BlockSpec((D, Q), lambda j: (0, 0)),
            pl.BlockSpec((KB, D), lambda j: (j, 0)),
            pl.BlockSpec((1, 1, KB), lambda j: (j, 0, 0)),
        ],
        out_specs=pl.BlockSpec((1, Q), lambda j: (0, 0)),
        out_shape=jax.ShapeDtypeStruct((1, Q), jnp.float32),
        scratch_shapes=[
            pltpu.VMEM((1, Q), jnp.float32),
            pltpu.VMEM((1, Q), jnp.float32),
        ],
    )(qT, data, lab_p)

    return (pred_row.reshape(Q, 1), jnp.asarray(0.0))
